# baseline (speedup 1.0000x reference)
import functools

import jax
import jax.numpy as jnp
from jax import lax
from jax.experimental import pallas as pl
from jax.experimental.pallas import tpu as pltpu
from jax.experimental.pallas import tpu_sc as plsc

N_TABLE = 1000000
M_IDX = 100000

NUM_CORES = 2
NUM_SUBCORES = 16
NUM_TILES = NUM_CORES * NUM_SUBCORES
CHUNK = 128
CHUNKS_PER_TILE = 25
PER_TILE = CHUNK * CHUNKS_PER_TILE
M_PAD = NUM_TILES * PER_TILE
VREGS_PER_CHUNK = CHUNK // 16


def _sc_body(lp_hbm, idx_hbm, mask_hbm, sel_hbm, out_hbm,
             idx_v, mask_v, lp_v, sel_v, row_v, sem_lp, sem_sel):
    c = lax.axis_index("c")
    s = lax.axis_index("s")
    wid = s * NUM_CORES + c

    pltpu.sync_copy(idx_hbm.at[wid], idx_v)
    pltpu.sync_copy(mask_hbm.at[wid], mask_v)

    def fire(j, _):
        pltpu.make_async_copy(lp_hbm.at[idx_v.at[j]], lp_v.at[j], sem_lp).start()
        pltpu.make_async_copy(sel_hbm.at[idx_v.at[j]], sel_v.at[j], sem_sel).start()
        return _

    lax.fori_loop(0, CHUNKS_PER_TILE, fire, None)

    def drain(j, _):
        pltpu.make_async_copy(lp_hbm.at[idx_v.at[j]], lp_v.at[j], sem_lp).wait()
        pltpu.make_async_copy(sel_hbm.at[idx_v.at[j]], sel_v.at[j], sem_sel).wait()
        return _

    lax.fori_loop(0, CHUNKS_PER_TILE, drain, None)

    def accum(i, carry):
        acc_lp, acc_ns, acc_sel = carry
        j = i // VREGS_PER_CHUNK
        v = (i % VREGS_PER_CHUNK) * 16
        m = mask_v[j, pl.ds(v, 16)]
        g = lp_v[j, pl.ds(v, 16)]
        sel = sel_v[j, pl.ds(v, 16)].astype(jnp.float32)
        return (acc_lp + g * m, acc_ns + m, acc_sel + sel * m)

    zero = jnp.zeros((16,), jnp.float32)
    acc_lp, acc_ns, acc_sel = lax.fori_loop(
        0, CHUNKS_PER_TILE * VREGS_PER_CHUNK, accum, (zero, zero, zero))

    row_v[pl.ds(0, 16)] = acc_lp
    row_v[pl.ds(16, 16)] = acc_ns
    row_v[pl.ds(32, 16)] = acc_sel
    for k in range(3, 8):
        row_v[pl.ds(k * 16, 16)] = zero
    pltpu.sync_copy(row_v, out_hbm.at[wid])


_sc_partials = functools.partial(
    pl.kernel,
    out_type=jax.ShapeDtypeStruct((NUM_TILES, 128), jnp.float32),
    mesh=plsc.VectorSubcoreMesh(
        core_axis_name="c", subcore_axis_name="s",
        num_cores=NUM_CORES, num_subcores=NUM_SUBCORES),
    scratch_types=[
        pltpu.VMEM((CHUNKS_PER_TILE, CHUNK), jnp.int32),
        pltpu.VMEM((CHUNKS_PER_TILE, CHUNK), jnp.float32),
        pltpu.VMEM((CHUNKS_PER_TILE, CHUNK), jnp.float32),
        pltpu.VMEM((CHUNKS_PER_TILE, CHUNK), jnp.int32),
        pltpu.VMEM((128,), jnp.float32),
        pltpu.SemaphoreType.DMA,
        pltpu.SemaphoreType.DMA,
    ],
)(_sc_body)


def _tc_reduce_body(x_ref, o_ref):
    x = x_ref[...]
    lane = lax.broadcasted_iota(jnp.int32, x.shape, 1)
    lp_sum = jnp.sum(jnp.where(lane < 16, x, 0.0))
    ns = jnp.sum(jnp.where((lane >= 16) & (lane < 32), x, 0.0))
    nc = jnp.sum(jnp.where((lane >= 32) & (lane < 48), x, 0.0))
    loss = jnp.where(ns > 0.0, -lp_sum / jnp.where(ns > 0.0, ns, 1.0), 0.0)
    olane = lax.broadcasted_iota(jnp.int32, (1, 128), 1)
    o_ref[...] = jnp.where(
        olane == 0, loss,
        jnp.where(olane == 1, ns, jnp.where(olane == 2, nc, 0.0)))


_tc_reduce = pl.pallas_call(
    _tc_reduce_body,
    out_shape=jax.ShapeDtypeStruct((1, 128), jnp.float32),
)


def kernel(candidate_logprobs, correct_candidate_idx, correct_is_nonpad,
           selected_fixes):
    idx = jnp.pad(correct_candidate_idx, (0, M_PAD - M_IDX))
    idx3 = idx.reshape(NUM_TILES, CHUNKS_PER_TILE, CHUNK)
    mask = jnp.pad(correct_is_nonpad.astype(jnp.float32), (0, M_PAD - M_IDX))
    mask3 = mask.reshape(NUM_TILES, CHUNKS_PER_TILE, CHUNK)
    sel_i32 = selected_fixes.astype(jnp.int32)

    partials = _sc_partials(candidate_logprobs, idx3, mask3, sel_i32)
    out = _tc_reduce(partials)

    loss = out[0, 0]
    num_samples = out[0, 1].astype(jnp.int32)
    num_correct = out[0, 2].astype(jnp.int32)
    return (loss, num_samples, num_correct)
